# baseline (device time: 248031 ns/iter reference)
import jax
import jax.numpy as jnp
from jax import lax
from jax.experimental import pallas as pl
from jax.experimental.pallas import tpu as pltpu

B = 4
S = 1024
S_HALF = 512
H = 16
D = 128
K = H * D
N = 4096
ROWS = 256
NC = B * S_HALF // ROWS


def kernel(O, Wo):
    Wo_b = Wo.astype(jnp.bfloat16)
    O_r = O.reshape(2 * B, S_HALF, H, D)

    def body(
        o_hbm,
        wo_ref,
        out_hbm,
        o_stage,
        o_bf,
        send_vmem,
        recv_vmem,
        acc_vmem,
        load_sem,
        store_sem,
        send_sems,
        recv_sems,
    ):
        my_x = lax.axis_index("x")
        my_y = lax.axis_index("y")
        my_z = lax.axis_index("z")
        nbr = (my_x, 1 - my_y, my_z)

        barrier = pltpu.get_barrier_semaphore()
        pl.semaphore_signal(
            barrier, inc=1, device_id=nbr, device_id_type=pl.DeviceIdType.MESH
        )
        pl.semaphore_wait(barrier, 1)

        def load_chunk(c, other_half, r0=0, nr=ROWS):
            half = (1 - my_y) if other_half else my_y
            j = 2 * (c // 2) + half
            s0 = (c % 2) * ROWS + r0
            for h in range(H):
                pltpu.make_async_copy(
                    o_hbm.at[j, pl.ds(s0, nr), h],
                    o_stage.at[pl.ds(0, nr), pl.ds(h * D, D)],
                    load_sem,
                ).start()
            for h in range(H):
                pltpu.make_async_copy(
                    o_hbm.at[j, pl.ds(s0, nr), h],
                    o_stage.at[pl.ds(0, nr), pl.ds(h * D, D)],
                    load_sem,
                ).wait()

        def partial_matmul(nr=ROWS):
            o_bf[0:nr, :] = o_stage[0:nr, :].astype(jnp.bfloat16)
            return jnp.dot(
                o_bf[0:nr, :], wo_ref[...], preferred_element_type=jnp.float32
            )

        HR = ROWS // 2
        PIECES = (
            [(0, 0, HR), (0, HR, HR)]
            + [(c, 0, ROWS) for c in range(1, NC - 1)]
            + [(NC - 1, 0, HR), (NC - 1, HR, HR)]
        )
        CHUNK_PIECES = {0: [0, 1], NC - 1: [len(PIECES) - 2, len(PIECES) - 1]}
        for c in range(1, NC - 1):
            CHUNK_PIECES[c] = [c + 1]

        rdmas = []
        for i, (c, r0, nr) in enumerate(PIECES):
            load_chunk(c, other_half=True, r0=r0, nr=nr)
            if i >= 4:
                rdmas[i - 4].wait_send()
            send_vmem[i % 4, 0:nr, :] = partial_matmul(nr).astype(jnp.bfloat16)
            rdma = pltpu.make_async_remote_copy(
                src_ref=send_vmem.at[i % 4, pl.ds(0, nr)],
                dst_ref=recv_vmem.at[c, pl.ds(r0, nr)],
                send_sem=send_sems.at[i],
                recv_sem=recv_sems.at[i],
                device_id=nbr,
                device_id_type=pl.DeviceIdType.MESH,
            )
            rdma.start()
            rdmas.append(rdma)

        for c in range(NC):
            load_chunk(c, other_half=False)
            p = partial_matmul()
            pieces = CHUNK_PIECES[c]
            if c < NC - 1:
                for i in pieces:
                    rdmas[i].wait_recv()
                acc_vmem[...] = p + recv_vmem[c].astype(jnp.float32)
                st = pltpu.make_async_copy(
                    acc_vmem, out_hbm.at[c // 2, pl.ds((c % 2) * ROWS, ROWS)],
                    store_sem,
                )
                st.start()
                st.wait()
            else:
                for i, r0 in zip(pieces, (0, HR)):
                    rdmas[i].wait_recv()
                    acc_vmem[r0 : r0 + HR, :] = p[r0 : r0 + HR, :] + recv_vmem[
                        c, r0 : r0 + HR, :
                    ].astype(jnp.float32)
                    st = pltpu.make_async_copy(
                        acc_vmem.at[pl.ds(r0, HR)],
                        out_hbm.at[c // 2, pl.ds((c % 2) * ROWS + r0, HR)],
                        store_sem,
                    )
                    st.start()
                    st.wait()

        for i in range(len(PIECES) - 4, len(PIECES)):
            rdmas[i].wait_send()

        pl.semaphore_signal(
            barrier, inc=1, device_id=nbr, device_id_type=pl.DeviceIdType.MESH
        )
        pl.semaphore_wait(barrier, 1)

    out = pl.pallas_call(
        body,
        out_shape=jax.ShapeDtypeStruct((B, S_HALF, N), jnp.float32),
        in_specs=[
            pl.BlockSpec(memory_space=pl.ANY),
            pl.BlockSpec(memory_space=pltpu.MemorySpace.VMEM),
        ],
        out_specs=pl.BlockSpec(memory_space=pl.ANY),
        scratch_shapes=[
            pltpu.VMEM((ROWS, K), jnp.float32),
            pltpu.VMEM((ROWS, K), jnp.bfloat16),
            pltpu.VMEM((4, ROWS, N), jnp.bfloat16),
            pltpu.VMEM((NC, ROWS, N), jnp.bfloat16),
            pltpu.VMEM((ROWS, N), jnp.float32),
            pltpu.SemaphoreType.DMA,
            pltpu.SemaphoreType.DMA,
            pltpu.SemaphoreType.DMA((NC + 2,)),
            pltpu.SemaphoreType.DMA((NC + 2,)),
        ],
        compiler_params=pltpu.CompilerParams(
            collective_id=0, vmem_limit_bytes=64 * 1024 * 1024
        ),
    )(O_r, Wo_b)
    return out
